# baseline (speedup 1.0000x reference)
import functools

import jax
import jax.numpy as jnp
from jax import lax
from jax.experimental import pallas as pl
from jax.experimental.pallas import tpu as pltpu
from jax.experimental.pallas import tpu_sc as plsc

B = 4096
L = 200
DIM = 128
NC, NS = 2, 16
NW = NC * NS
BPW = B // NW
HALF = L // 2
NG = DIM // 16


def _sc_gather_mean(reco_idx, search_idx, open_idx, user_id,
                    reco_table, search_table, user_table):
    mesh = plsc.VectorSubcoreMesh(core_axis_name="c", subcore_axis_name="s",
                                  num_cores=NC, num_subcores=NS)
    out_t = (jax.ShapeDtypeStruct((B, DIM), jnp.float32),) * 4
    scratch = [
        pltpu.VMEM((BPW * L,), jnp.int32),
        pltpu.VMEM((L, DIM), jnp.float32),
        pltpu.VMEM((L, DIM), jnp.float32),
        pltpu.VMEM((L, DIM), jnp.float32),
        pltpu.VMEM((L, DIM), jnp.float32),
        pltpu.VMEM((16, DIM), jnp.float32),
        pltpu.VMEM((BPW,), jnp.int32),
        pltpu.SemaphoreType.DMA,
        pltpu.SemaphoreType.DMA,
        pltpu.SemaphoreType.DMA,
        pltpu.SemaphoreType.DMA,
        pltpu.SemaphoreType.DMA,
        pltpu.SemaphoreType.DMA,
        pltpu.SemaphoreType.DMA,
        pltpu.SemaphoreType.DMA,
        pltpu.SemaphoreType.DMA,
    ]
    C0 = 104
    C1 = L - C0

    @functools.partial(pl.kernel, out_type=out_t, mesh=mesh, scratch_types=scratch)
    def k(reco_idx_h, search_idx_h, open_idx_h, uid_h, reco_t, search_t, user_t,
          out_r, out_s, out_o, out_u, idx_v, rowsA_v, rowsB_v, rowsC_v, rowsD_v,
          ring_v, uidx_v, semA0, semA1, semB0, semB1, semC0, semC1, semD0,
          semD1, semW):
        wid = lax.axis_index("s") * NC + lax.axis_index("c")
        base = wid * BPW
        slots = ((rowsA_v, semA0, semA1),
                 (rowsB_v, semB0, semB1),
                 (rowsC_v, semC0, semC1),
                 (rowsD_v, semD0, semD1))
        NSLOT = len(slots)

        def pool_one(idx_h, table, out):
            pltpu.sync_copy(idx_h.at[pl.ds(base * L, BPW * L)], idx_v)

            def issue(b, slot):
                buf, semA, semB = slot
                off = pl.multiple_of(b * L, 8)
                pltpu.async_copy(table.at[idx_v.at[pl.ds(off, C0)]],
                                 buf.at[pl.ds(0, C0)], semA)
                pltpu.async_copy(table.at[idx_v.at[pl.ds(off + C0, C1)]],
                                 buf.at[pl.ds(C0, C1)], semB)

            def wait_chunk(buf, off, n, sem):
                pltpu.make_async_copy(table.at[idx_v.at[pl.ds(0, n)]],
                                      buf.at[pl.ds(off, n)], sem).wait()

            def accum_part(buf, off, n, accs):
                def acc_body(r, accs):
                    new = []
                    for j in range(NG):
                        r0 = buf[off + 4 * r, pl.ds(j * 16, 16)]
                        r1 = buf[off + 4 * r + 1, pl.ds(j * 16, 16)]
                        r2 = buf[off + 4 * r + 2, pl.ds(j * 16, 16)]
                        r3 = buf[off + 4 * r + 3, pl.ds(j * 16, 16)]
                        new.append(accs[j] + ((r0 + r1) + (r2 + r3)))
                    return tuple(new)

                return lax.fori_loop(0, n // 4, acc_body, accs)

            zeros = tuple(jnp.zeros((16,), jnp.float32) for _ in range(NG))

            def accum(rloc, p):
                buf, semA, semB = slots[p]
                wait_chunk(buf, 0, C0, semA)
                accs = accum_part(buf, 0, C0, zeros)
                wait_chunk(buf, C0, C1, semB)
                accs = accum_part(buf, C0, C1, accs)
                for j in range(NG):
                    ring_v[rloc, pl.ds(j * 16, 16)] = accs[j] * (1.0 / L)

            for p in range(NSLOT):
                issue(p, slots[p])

            def body(i, carry):
                b0 = NSLOT * i
                phase = jnp.bitwise_and(i, 3)

                @pl.when(jnp.logical_and(phase == 0, i > 0))
                def _():
                    pltpu.make_async_copy(ring_v, out.at[pl.ds(base, 16), :],
                                          semW).wait()

                for p in range(NSLOT):
                    accum(NSLOT * phase + p, p)

                    @pl.when(b0 + p + NSLOT < BPW)
                    def _():
                        issue(b0 + p + NSLOT, slots[p])

                @pl.when(phase == 3)
                def _():
                    pltpu.async_copy(
                        ring_v,
                        out.at[pl.ds(pl.multiple_of(base + b0 - 12, 8), 16), :],
                        semW)
                return carry

            lax.fori_loop(0, BPW // NSLOT, body, 0)
            pltpu.make_async_copy(ring_v, out.at[pl.ds(base, 16), :],
                                  semW).wait()

        pool_one(reco_idx_h, reco_t, out_r)
        pool_one(search_idx_h, search_t, out_s)
        pool_one(open_idx_h, search_t, out_o)

        pltpu.sync_copy(uid_h.at[pl.ds(base, BPW)], uidx_v)
        pltpu.async_copy(user_t.at[uidx_v], rowsA_v.at[pl.ds(0, BPW)],
                         semA0).wait()
        pltpu.sync_copy(rowsA_v.at[pl.ds(0, BPW)],
                        out_u.at[pl.ds(base, BPW), :])

    return k(reco_idx, search_idx, open_idx, user_id,
             reco_table, search_table, user_table)


def _tc_mlp(rm, sm, om, ur, t128, w1r, w1s, w1o, w1u, w1t, b1, w2p, b2p):
    def body(r_ref, s_ref, o_ref, u_ref, t_ref, wr_ref, ws_ref, wo_ref, wu_ref,
             wt_ref, b1_ref, w2_ref, b2_ref, out_ref):
        h = (jnp.dot(r_ref[...], wr_ref[...], preferred_element_type=jnp.float32)
             + jnp.dot(s_ref[...], ws_ref[...], preferred_element_type=jnp.float32)
             + jnp.dot(o_ref[...], wo_ref[...], preferred_element_type=jnp.float32)
             + jnp.dot(u_ref[...], wu_ref[...], preferred_element_type=jnp.float32)
             + jnp.dot(t_ref[...], wt_ref[...], preferred_element_type=jnp.float32)
             + b1_ref[...])
        h = jnp.where(h >= 0, h, 0.01 * h)
        out_ref[...] = (jnp.dot(h, w2_ref[...], preferred_element_type=jnp.float32)
                        + b2_ref[...])

    return pl.pallas_call(
        body,
        out_shape=jax.ShapeDtypeStruct((B, DIM), jnp.float32),
    )(rm, sm, om, ur, t128, w1r, w1s, w1o, w1u, w1t, b1, w2p, b2p)


def kernel(reco_history, search_history, open_search_history, time_features,
           user_id, reco_table, search_table, user_table, W1, b1, W2, b2):
    ri = reco_history.astype(jnp.int32).reshape(B * L)
    si = search_history.astype(jnp.int32).reshape(B * L)
    oi = open_search_history.astype(jnp.int32).reshape(B * L)
    uid = user_id.astype(jnp.int32)

    rm, sm, om, ur = _sc_gather_mean(ri, si, oi, uid,
                                     reco_table, search_table, user_table)

    t128 = jnp.pad(time_features, ((0, 0), (0, DIM - 6)))
    w1r = W1[0:128]
    w1s = W1[128:256]
    w1o = W1[256:384]
    w1u = W1[384:512]
    w1t = jnp.pad(W1[512:518], ((0, DIM - 6), (0, 0)))
    b1r = b1.reshape(1, DIM)
    w2p = jnp.pad(W2, ((0, 0), (0, DIM - 2)))
    b2p = jnp.pad(b2, (0, DIM - 2)).reshape(1, DIM)

    out = _tc_mlp(rm, sm, om, ur, t128, w1r, w1s, w1o, w1u, w1t, b1r, w2p, b2p)
    return out[:, :2]


# speedup vs baseline: 1.0344x; 1.0344x over previous
import functools

import jax
import jax.numpy as jnp
from jax import lax
from jax.experimental import pallas as pl
from jax.experimental.pallas import tpu as pltpu
from jax.experimental.pallas import tpu_sc as plsc

B = 4096
L = 200
DIM = 128
NC, NS = 2, 16
NW = NC * NS
BPW = B // NW
HALF = L // 2
NG = DIM // 16


def _sc_gather_mean(reco_idx, search_idx, open_idx, user_id,
                    reco_table, search_table, user_table):
    mesh = plsc.VectorSubcoreMesh(core_axis_name="c", subcore_axis_name="s",
                                  num_cores=NC, num_subcores=NS)
    out_t = (jax.ShapeDtypeStruct((B, DIM), jnp.float32),) * 4
    scratch = [
        pltpu.VMEM((2 * BPW, HALF), jnp.int32),
        pltpu.VMEM((L, DIM), jnp.float32),
        pltpu.VMEM((L, DIM), jnp.float32),
        pltpu.VMEM((L, DIM), jnp.float32),
        pltpu.VMEM((BPW, DIM), jnp.float32),
        pltpu.VMEM((BPW,), jnp.int32),
        pltpu.SemaphoreType.DMA,
        pltpu.SemaphoreType.DMA,
        pltpu.SemaphoreType.DMA,
        pltpu.SemaphoreType.DMA,
        pltpu.SemaphoreType.DMA,
        pltpu.SemaphoreType.DMA,
        pltpu.SemaphoreType.DMA,
    ]

    @functools.partial(pl.kernel, out_type=out_t, mesh=mesh, scratch_types=scratch)
    def k(reco_idx_h, search_idx_h, open_idx_h, uid_h, reco_t, search_t, user_t,
          out_r, out_s, out_o, out_u, idx_v, rowsA_v, rowsB_v, rowsC_v, res_v,
          uidx_v, semA0, semA1, semB0, semB1, semC0, semC1, semR):
        wid = lax.axis_index("s") * NC + lax.axis_index("c")
        base = wid * BPW
        slots = ((rowsA_v, semA0, semA1),
                 (rowsB_v, semB0, semB1),
                 (rowsC_v, semC0, semC1))
        NSLOT = len(slots)

        def pool_one(idx_h, table, out, prev_out):
            pltpu.sync_copy(idx_h.at[pl.ds(base * 2, 2 * BPW), :], idx_v)

            def issue(b, slot):
                buf, semA, semB = slot
                pltpu.async_copy(table.at[idx_v.at[2 * b]],
                                 buf.at[pl.ds(0, HALF)], semA)
                pltpu.async_copy(table.at[idx_v.at[2 * b + 1]],
                                 buf.at[pl.ds(HALF, HALF)], semB)

            def wait_chunk(buf, off, sem):
                pltpu.make_async_copy(table.at[idx_v.at[0]],
                                      buf.at[pl.ds(off, HALF)], sem).wait()

            def accum_half(buf, off, accs):
                def acc_body(r, accs):
                    new = []
                    for j in range(NG):
                        r0 = buf[off + 4 * r, pl.ds(j * 16, 16)]
                        r1 = buf[off + 4 * r + 1, pl.ds(j * 16, 16)]
                        r2 = buf[off + 4 * r + 2, pl.ds(j * 16, 16)]
                        r3 = buf[off + 4 * r + 3, pl.ds(j * 16, 16)]
                        new.append(accs[j] + ((r0 + r1) + (r2 + r3)))
                    return tuple(new)

                return lax.fori_loop(0, HALF // 4, acc_body, accs)

            zeros = tuple(jnp.zeros((16,), jnp.float32) for _ in range(NG))

            def accum(b, slot):
                buf, semA, semB = slot
                wait_chunk(buf, 0, semA)
                accs = accum_half(buf, 0, zeros)
                wait_chunk(buf, HALF, semB)
                accs = accum_half(buf, HALF, accs)
                for j in range(NG):
                    res_v[b, pl.ds(j * 16, 16)] = accs[j] * (1.0 / L)

            for p in range(NSLOT):
                issue(p, slots[p])
            if prev_out is not None:
                pltpu.make_async_copy(res_v, prev_out.at[pl.ds(base, BPW), :],
                                      semR).wait()

            def body(i, carry):
                b0 = NSLOT * i
                for p in range(NSLOT):
                    accum(b0 + p, slots[p])

                    @pl.when(b0 + p + NSLOT < BPW)
                    def _():
                        issue(b0 + p + NSLOT, slots[p])
                return carry

            nfull = BPW // NSLOT
            lax.fori_loop(0, nfull, body, 0)
            for b in range(nfull * NSLOT, BPW):
                accum(b, slots[b % NSLOT])
            pltpu.async_copy(res_v, out.at[pl.ds(base, BPW), :], semR)

        pool_one(reco_idx_h, reco_t, out_r, None)
        pool_one(search_idx_h, search_t, out_s, out_r)
        pool_one(open_idx_h, search_t, out_o, out_s)

        pltpu.sync_copy(uid_h.at[pl.ds(base, BPW)], uidx_v)
        pltpu.make_async_copy(res_v, out_o.at[pl.ds(base, BPW), :], semR).wait()
        pltpu.async_copy(user_t.at[uidx_v], res_v, semA0).wait()
        pltpu.sync_copy(res_v, out_u.at[pl.ds(base, BPW), :])

    return k(reco_idx, search_idx, open_idx, user_id,
             reco_table, search_table, user_table)


def _tc_mlp(rm, sm, om, ur, t128, w1r, w1s, w1o, w1u, w1t, b1, w2p, b2p):
    def body(r_ref, s_ref, o_ref, u_ref, t_ref, wr_ref, ws_ref, wo_ref, wu_ref,
             wt_ref, b1_ref, w2_ref, b2_ref, out_ref):
        h = (jnp.dot(r_ref[...], wr_ref[...], preferred_element_type=jnp.float32)
             + jnp.dot(s_ref[...], ws_ref[...], preferred_element_type=jnp.float32)
             + jnp.dot(o_ref[...], wo_ref[...], preferred_element_type=jnp.float32)
             + jnp.dot(u_ref[...], wu_ref[...], preferred_element_type=jnp.float32)
             + jnp.dot(t_ref[...], wt_ref[...], preferred_element_type=jnp.float32)
             + b1_ref[...])
        h = jnp.where(h >= 0, h, 0.01 * h)
        out_ref[...] = (jnp.dot(h, w2_ref[...], preferred_element_type=jnp.float32)
                        + b2_ref[...])

    return pl.pallas_call(
        body,
        out_shape=jax.ShapeDtypeStruct((B, DIM), jnp.float32),
    )(rm, sm, om, ur, t128, w1r, w1s, w1o, w1u, w1t, b1, w2p, b2p)


def kernel(reco_history, search_history, open_search_history, time_features,
           user_id, reco_table, search_table, user_table, W1, b1, W2, b2):
    ri = reco_history.astype(jnp.int32).reshape(2 * B, HALF)
    si = search_history.astype(jnp.int32).reshape(2 * B, HALF)
    oi = open_search_history.astype(jnp.int32).reshape(2 * B, HALF)
    uid = user_id.astype(jnp.int32)

    rm, sm, om, ur = _sc_gather_mean(ri, si, oi, uid,
                                     reco_table, search_table, user_table)

    t128 = jnp.pad(time_features, ((0, 0), (0, DIM - 6)))
    w1r = W1[0:128]
    w1s = W1[128:256]
    w1o = W1[256:384]
    w1u = W1[384:512]
    w1t = jnp.pad(W1[512:518], ((0, DIM - 6), (0, 0)))
    b1r = b1.reshape(1, DIM)
    w2p = jnp.pad(W2, ((0, 0), (0, DIM - 2)))
    b2p = jnp.pad(b2, (0, DIM - 2)).reshape(1, DIM)

    out = _tc_mlp(rm, sm, om, ur, t128, w1r, w1s, w1o, w1u, w1t, b1r, w2p, b2p)
    return out[:, :2]
